# Initial kernel scaffold; baseline (speedup 1.0000x reference)
import jax
import jax.numpy as jnp
from jax.experimental import pallas as pl


def kernel(x, edge_index, edge_attr, batch, enc_Wn, enc_bn, enc_We, enc_be, W_msg, b_msg, W_node, b_node, W_pre, b_pre, W_out, b_out):
    raise NotImplementedError("write your pallas kernel here")


# speedup vs baseline: 2.4194x; 2.4194x over previous
import functools

import jax
import jax.numpy as jnp
from jax import lax
from jax.experimental import pallas as pl
from jax.experimental.pallas import tpu as pltpu
from jax.experimental.pallas import tpu_sc as plsc

NN = 10000
NE = 320000
DE = 16
H = 128
NG = 64
NL = 4

NC, NS = 2, 16
NW = NC * NS
EPW = NE // NW
CH = 80
NCHUNK = EPW // CH
RPT = 640
AGG_PAD = NS * RPT

F32 = jnp.float32

_MESH = dict(core_axis_name="c", subcore_axis_name="s",
             num_cores=NC, num_subcores=NS)


def _linear_body(x_ref, w_ref, b_ref, o_ref, *, relu):
    y = jnp.dot(x_ref[...], w_ref[...], preferred_element_type=F32) + b_ref[...]
    if relu:
        y = jnp.maximum(y, 0.0)
    o_ref[...] = y


def _linear(x, w, b, *, relu, bm):
    m, k = x.shape
    n = w.shape[1]
    return pl.pallas_call(
        functools.partial(_linear_body, relu=relu),
        grid=(m // bm,),
        in_specs=[
            pl.BlockSpec((bm, k), lambda i: (i, 0)),
            pl.BlockSpec((k, n), lambda i: (0, 0)),
            pl.BlockSpec((1, n), lambda i: (0, 0)),
        ],
        out_specs=pl.BlockSpec((bm, n), lambda i: (i, 0)),
        out_shape=jax.ShapeDtypeStruct((m, n), F32),
    )(x, w, b.reshape(1, n))


def _ab_body(h_ref, w1_ref, w2_ref, b1_ref, a_ref, b_ref):
    hb = h_ref[...]
    a_ref[...] = jnp.dot(hb, w1_ref[...], preferred_element_type=F32) + b1_ref[...]
    b_ref[...] = jnp.dot(hb, w2_ref[...], preferred_element_type=F32)


def _ab(h, w1, w2, bmsg, bm=2000):
    return pl.pallas_call(
        _ab_body,
        grid=(NN // bm,),
        in_specs=[
            pl.BlockSpec((bm, H), lambda i: (i, 0)),
            pl.BlockSpec((H, H), lambda i: (0, 0)),
            pl.BlockSpec((H, H), lambda i: (0, 0)),
            pl.BlockSpec((1, H), lambda i: (0, 0)),
        ],
        out_specs=[pl.BlockSpec((bm, H), lambda i: (i, 0))] * 2,
        out_shape=[jax.ShapeDtypeStruct((NN, H), F32)] * 2,
    )(h, w1, w2, bmsg.reshape(1, H))


def _msg_body(e_ref, g1_ref, g2_ref, w3_ref, o_ref):
    y = jnp.dot(e_ref[...], w3_ref[...], preferred_element_type=F32)
    o_ref[...] = jnp.maximum(y + g1_ref[...] + g2_ref[...], 0.0)


def _msg(e, g1, g2, w3, be=1280):
    return pl.pallas_call(
        _msg_body,
        grid=(NE // be,),
        in_specs=[
            pl.BlockSpec((be, H), lambda i: (i, 0)),
            pl.BlockSpec((be, H), lambda i: (i, 0)),
            pl.BlockSpec((be, H), lambda i: (i, 0)),
            pl.BlockSpec((H, H), lambda i: (0, 0)),
        ],
        out_specs=pl.BlockSpec((be, H), lambda i: (i, 0)),
        out_shape=jax.ShapeDtypeStruct((NE, H), F32),
    )(e, g1, g2, w3)


def _node_body(h_ref, a0_ref, a1_ref, wn1_ref, wn2_ref, b_ref, o_ref):
    agg = a0_ref[0] + a1_ref[0]
    y = jnp.dot(h_ref[...], wn1_ref[...], preferred_element_type=F32)
    y = y + jnp.dot(agg, wn2_ref[...], preferred_element_type=F32) + b_ref[...]
    o_ref[...] = jnp.maximum(y, 0.0)


def _node(h, agg, wn1, wn2, bnode, bm=2000):
    return pl.pallas_call(
        _node_body,
        grid=(NN // bm,),
        in_specs=[
            pl.BlockSpec((bm, H), lambda i: (i, 0)),
            pl.BlockSpec((1, bm, H), lambda i: (0, i, 0)),
            pl.BlockSpec((1, bm, H), lambda i: (1, i, 0)),
            pl.BlockSpec((H, H), lambda i: (0, 0)),
            pl.BlockSpec((H, H), lambda i: (0, 0)),
            pl.BlockSpec((1, H), lambda i: (0, 0)),
        ],
        out_specs=pl.BlockSpec((bm, H), lambda i: (i, 0)),
        out_shape=jax.ShapeDtypeStruct((NN, H), F32),
    )(h, agg, agg, wn1, wn2, bnode.reshape(1, H))


_BE_RD = 1280


def _readout_body(e_ref, g1_ref, g2_ref, src_ref, batch_ref, w3_ref, wpre_ref,
                  bpre_ref, wout_ref, bout_ref, o_ref, acc_ref, cnt_ref,
                  lt_ref, le_ref):
    i = pl.program_id(0)

    @pl.when(i == 0)
    def _():
        acc_ref[...] = jnp.zeros_like(acc_ref)
        cnt_ref[...] = jnp.zeros_like(cnt_ref)
        brow = batch_ref[...]
        gcol = lax.broadcasted_iota(jnp.int32, (NG, 1), 0)
        lt_ref[...] = jnp.sum((brow < gcol).astype(jnp.int32), axis=1,
                              keepdims=True)
        le_ref[...] = jnp.sum((brow <= gcol).astype(jnp.int32), axis=1,
                              keepdims=True)

    m = jnp.dot(e_ref[...], w3_ref[...], preferred_element_type=F32)
    m = jnp.maximum(m + g1_ref[...] + g2_ref[...], 0.0)
    phi = jnp.maximum(
        jnp.dot(m, wpre_ref[...], preferred_element_type=F32) + bpre_ref[...], 0.0)
    srow = src_ref[0]
    onehot = ((srow >= lt_ref[...]) & (srow < le_ref[...])).astype(F32)
    acc_ref[...] += jnp.dot(onehot, phi, preferred_element_type=F32)
    cnt_ref[...] += jnp.dot(onehot, jnp.ones_like(phi), preferred_element_type=F32)

    @pl.when(i == pl.num_programs(0) - 1)
    def _():
        pooled = acc_ref[...] / jnp.maximum(cnt_ref[...], 1.0)
        o_ref[...] = (jnp.dot(pooled, wout_ref[...], preferred_element_type=F32)
                      + bout_ref[...])


def _readout(e, g1, g2, src3, batch2, w3, wpre, bpre, wout, bout):
    nblk = NE // _BE_RD
    return pl.pallas_call(
        _readout_body,
        grid=(nblk,),
        in_specs=[
            pl.BlockSpec((_BE_RD, H), lambda i: (i, 0)),
            pl.BlockSpec((_BE_RD, H), lambda i: (i, 0)),
            pl.BlockSpec((_BE_RD, H), lambda i: (i, 0)),
            pl.BlockSpec((1, 1, _BE_RD), lambda i: (i, 0, 0)),
            pl.BlockSpec((1, NN), lambda i: (0, 0)),
            pl.BlockSpec((H, H), lambda i: (0, 0)),
            pl.BlockSpec((H, H), lambda i: (0, 0)),
            pl.BlockSpec((1, H), lambda i: (0, 0)),
            pl.BlockSpec((H, H), lambda i: (0, 0)),
            pl.BlockSpec((1, H), lambda i: (0, 0)),
        ],
        out_specs=pl.BlockSpec((NG, H), lambda i: (0, 0)),
        out_shape=jax.ShapeDtypeStruct((NG, H), F32),
        scratch_shapes=[
            pltpu.VMEM((NG, H), F32),
            pltpu.VMEM((NG, H), F32),
            pltpu.VMEM((NG, 1), jnp.int32),
            pltpu.VMEM((NG, 1), jnp.int32),
        ],
    )(e, g1, g2, src3, batch2, w3, wpre, bpre.reshape(1, H), wout,
      bout.reshape(1, H))


def _sc_gather_body(src_ref, dst_ref, a_ref, b_ref, g1_ref, g2_ref,
                    idx_ref, rows_ref, sem):
    c = lax.axis_index("c")
    s = lax.axis_index("s")
    base = (s * NC + c) * EPW

    def chunk(i, carry):
        off = base + i * CH
        pltpu.sync_copy(src_ref.at[pl.ds(off, CH)], idx_ref)
        pltpu.async_copy(a_ref.at[idx_ref], rows_ref, sem).wait()
        pltpu.sync_copy(rows_ref, g1_ref.at[pl.ds(off, CH)])
        pltpu.sync_copy(dst_ref.at[pl.ds(off, CH)], idx_ref)
        pltpu.async_copy(b_ref.at[idx_ref], rows_ref, sem).wait()
        pltpu.sync_copy(rows_ref, g2_ref.at[pl.ds(off, CH)])
        return carry

    lax.fori_loop(0, NCHUNK, chunk, 0)


@functools.lru_cache(maxsize=None)
def _sc_gather_kernel():
    return pl.kernel(
        _sc_gather_body,
        out_type=(jax.ShapeDtypeStruct((NE, H), F32),) * 2,
        mesh=plsc.VectorSubcoreMesh(**_MESH),
        scratch_types=[
            pltpu.VMEM((CH,), jnp.int32),
            pltpu.VMEM((CH, H), F32),
            pltpu.SemaphoreType.DMA,
        ],
    )


def _sc_gather(src, dst, a, b):
    return _sc_gather_kernel()(src, dst, a, b)


def _sc_scatter_body(m_ref, dst_ref, zeros_ref, out_ref, idx_ref, m_v, agg_sh):
    c = lax.axis_index("c")
    s = lax.axis_index("s")
    pltpu.sync_copy(zeros_ref, agg_sh.at[pl.ds(s * RPT, RPT)])
    plsc.subcore_barrier()

    base = (c * NS + s) * EPW

    def chunk(i, carry):
        off = base + i * CH
        pltpu.sync_copy(dst_ref.at[pl.ds(off, CH)], idx_ref)
        pltpu.sync_copy(m_ref.at[pl.ds(off, CH)], m_v)
        pltpu.sync_copy(m_v, agg_sh.at[idx_ref], add=True)
        return carry

    lax.fori_loop(0, NCHUNK, chunk, 0)
    plsc.subcore_barrier()
    pltpu.sync_copy(agg_sh.at[pl.ds(s * RPT, RPT)],
                    out_ref.at[c, pl.ds(s * RPT, RPT)])


@functools.lru_cache(maxsize=None)
def _sc_scatter_kernel():
    return pl.kernel(
        _sc_scatter_body,
        out_type=jax.ShapeDtypeStruct((NC, AGG_PAD, H), F32),
        mesh=plsc.VectorSubcoreMesh(**_MESH),
        scratch_types=[
            pltpu.VMEM((CH,), jnp.int32),
            pltpu.VMEM((CH, H), F32),
            pltpu.VMEM_SHARED((AGG_PAD, H), F32),
        ],
    )


def _sc_scatter(m, dst, zeros_rt):
    return _sc_scatter_kernel()(m, dst, zeros_rt)


def kernel(x, edge_index, edge_attr, batch, enc_Wn, enc_bn, enc_We, enc_be,
           W_msg, b_msg, W_node, b_node, W_pre, b_pre, W_out, b_out):
    src = edge_index[0]
    dst = edge_index[1]

    h = _linear(x, enc_Wn, enc_bn, relu=True, bm=2000)
    e = _linear(edge_attr, enc_We, enc_be, relu=True, bm=2000)
    zeros_rt = jnp.zeros((RPT, H), F32)

    out = None
    for l in range(NL):
        w1 = W_msg[l, :H]
        w2 = W_msg[l, H:2 * H]
        w3 = W_msg[l, 2 * H:]
        a, b = _ab(h, w1, w2, b_msg[l])
        g1, g2 = _sc_gather(src, dst, a, b)
        if l < NL - 1:
            e = _msg(e, g1, g2, w3)
            agg = _sc_scatter(e, dst, zeros_rt)
            h = _node(h, agg, W_node[l, :H], W_node[l, H:], b_node[l])
        else:
            out = _readout(e, g1, g2, src.reshape(NE // _BE_RD, 1, _BE_RD),
                           batch.reshape(1, NN), w3, W_pre, b_pre, W_out, b_out)
    return out


# speedup vs baseline: 3.7528x; 1.5511x over previous
import functools

import jax
import jax.numpy as jnp
from jax import lax
from jax.experimental import pallas as pl
from jax.experimental.pallas import tpu as pltpu
from jax.experimental.pallas import tpu_sc as plsc

NN = 10000
NE = 320000
DE = 16
H = 128
NG = 64
NL = 4

NC, NS = 2, 16
NW = NC * NS
EPW = NE // NW
CH = 80
NCHUNK = EPW // CH
NPAIR = NCHUNK // 2
RPT = 640
AGG_PAD = NS * RPT

F32 = jnp.float32

_MESH = dict(core_axis_name="c", subcore_axis_name="s",
             num_cores=NC, num_subcores=NS)


def _linear_body(x_ref, w_ref, b_ref, o_ref, *, relu):
    y = jnp.dot(x_ref[...], w_ref[...], preferred_element_type=F32) + b_ref[...]
    if relu:
        y = jnp.maximum(y, 0.0)
    o_ref[...] = y


def _linear(x, w, b, *, relu, bm):
    m, k = x.shape
    n = w.shape[1]
    return pl.pallas_call(
        functools.partial(_linear_body, relu=relu),
        grid=(m // bm,),
        in_specs=[
            pl.BlockSpec((bm, k), lambda i: (i, 0)),
            pl.BlockSpec((k, n), lambda i: (0, 0)),
            pl.BlockSpec((1, n), lambda i: (0, 0)),
        ],
        out_specs=pl.BlockSpec((bm, n), lambda i: (i, 0)),
        out_shape=jax.ShapeDtypeStruct((m, n), F32),
    )(x, w, b.reshape(1, n))


def _ab_body(h_ref, w1_ref, w2_ref, b1_ref, a_ref, b_ref):
    hb = h_ref[...]
    a_ref[...] = jnp.dot(hb, w1_ref[...], preferred_element_type=F32) + b1_ref[...]
    b_ref[...] = jnp.dot(hb, w2_ref[...], preferred_element_type=F32)


def _ab(h, w1, w2, bmsg, bm=2000):
    return pl.pallas_call(
        _ab_body,
        grid=(NN // bm,),
        in_specs=[
            pl.BlockSpec((bm, H), lambda i: (i, 0)),
            pl.BlockSpec((H, H), lambda i: (0, 0)),
            pl.BlockSpec((H, H), lambda i: (0, 0)),
            pl.BlockSpec((1, H), lambda i: (0, 0)),
        ],
        out_specs=[pl.BlockSpec((bm, H), lambda i: (i, 0))] * 2,
        out_shape=[jax.ShapeDtypeStruct((NN, H), F32)] * 2,
    )(h, w1, w2, bmsg.reshape(1, H))


def _msg_body(e_ref, g1_ref, g2_ref, w3_ref, o_ref):
    y = jnp.dot(e_ref[...], w3_ref[...], preferred_element_type=F32)
    o_ref[...] = jnp.maximum(y + g1_ref[...] + g2_ref[...], 0.0)


def _msg(e, g1, g2, w3, be=1280):
    return pl.pallas_call(
        _msg_body,
        grid=(NE // be,),
        in_specs=[
            pl.BlockSpec((be, H), lambda i: (i, 0)),
            pl.BlockSpec((be, H), lambda i: (i, 0)),
            pl.BlockSpec((be, H), lambda i: (i, 0)),
            pl.BlockSpec((H, H), lambda i: (0, 0)),
        ],
        out_specs=pl.BlockSpec((be, H), lambda i: (i, 0)),
        out_shape=jax.ShapeDtypeStruct((NE, H), F32),
    )(e, g1, g2, w3)


def _node_body(h_ref, a0_ref, a1_ref, wn1_ref, wn2_ref, b_ref, o_ref):
    agg = a0_ref[0] + a1_ref[0]
    y = jnp.dot(h_ref[...], wn1_ref[...], preferred_element_type=F32)
    y = y + jnp.dot(agg, wn2_ref[...], preferred_element_type=F32) + b_ref[...]
    o_ref[...] = jnp.maximum(y, 0.0)


def _node(h, agg, wn1, wn2, bnode, bm=2000):
    return pl.pallas_call(
        _node_body,
        grid=(NN // bm,),
        in_specs=[
            pl.BlockSpec((bm, H), lambda i: (i, 0)),
            pl.BlockSpec((1, bm, H), lambda i: (0, i, 0)),
            pl.BlockSpec((1, bm, H), lambda i: (1, i, 0)),
            pl.BlockSpec((H, H), lambda i: (0, 0)),
            pl.BlockSpec((H, H), lambda i: (0, 0)),
            pl.BlockSpec((1, H), lambda i: (0, 0)),
        ],
        out_specs=pl.BlockSpec((bm, H), lambda i: (i, 0)),
        out_shape=jax.ShapeDtypeStruct((NN, H), F32),
    )(h, agg, agg, wn1, wn2, bnode.reshape(1, H))


_BE_RD = 1280


def _readout_body(e_ref, g1_ref, g2_ref, src_ref, batch_ref, w3_ref, wpre_ref,
                  bpre_ref, wout_ref, bout_ref, o_ref, acc_ref, cnt_ref,
                  lt_ref, le_ref):
    i = pl.program_id(0)

    @pl.when(i == 0)
    def _():
        acc_ref[...] = jnp.zeros_like(acc_ref)
        cnt_ref[...] = jnp.zeros_like(cnt_ref)
        brow = batch_ref[...]
        gcol = lax.broadcasted_iota(jnp.int32, (NG, 1), 0)
        lt_ref[...] = jnp.sum((brow < gcol).astype(jnp.int32), axis=1,
                              keepdims=True)
        le_ref[...] = jnp.sum((brow <= gcol).astype(jnp.int32), axis=1,
                              keepdims=True)

    m = jnp.dot(e_ref[...], w3_ref[...], preferred_element_type=F32)
    m = jnp.maximum(m + g1_ref[...] + g2_ref[...], 0.0)
    phi = jnp.maximum(
        jnp.dot(m, wpre_ref[...], preferred_element_type=F32) + bpre_ref[...], 0.0)
    srow = src_ref[0]
    onehot = ((srow >= lt_ref[...]) & (srow < le_ref[...])).astype(F32)
    acc_ref[...] += jnp.dot(onehot, phi, preferred_element_type=F32)
    cnt_ref[...] += jnp.dot(onehot, jnp.ones_like(phi), preferred_element_type=F32)

    @pl.when(i == pl.num_programs(0) - 1)
    def _():
        pooled = acc_ref[...] / jnp.maximum(cnt_ref[...], 1.0)
        o_ref[...] = (jnp.dot(pooled, wout_ref[...], preferred_element_type=F32)
                      + bout_ref[...])


def _readout(e, g1, g2, src3, batch2, w3, wpre, bpre, wout, bout):
    nblk = NE // _BE_RD
    return pl.pallas_call(
        _readout_body,
        grid=(nblk,),
        in_specs=[
            pl.BlockSpec((_BE_RD, H), lambda i: (i, 0)),
            pl.BlockSpec((_BE_RD, H), lambda i: (i, 0)),
            pl.BlockSpec((_BE_RD, H), lambda i: (i, 0)),
            pl.BlockSpec((1, 1, _BE_RD), lambda i: (i, 0, 0)),
            pl.BlockSpec((1, NN), lambda i: (0, 0)),
            pl.BlockSpec((H, H), lambda i: (0, 0)),
            pl.BlockSpec((H, H), lambda i: (0, 0)),
            pl.BlockSpec((1, H), lambda i: (0, 0)),
            pl.BlockSpec((H, H), lambda i: (0, 0)),
            pl.BlockSpec((1, H), lambda i: (0, 0)),
        ],
        out_specs=pl.BlockSpec((NG, H), lambda i: (0, 0)),
        out_shape=jax.ShapeDtypeStruct((NG, H), F32),
        scratch_shapes=[
            pltpu.VMEM((NG, H), F32),
            pltpu.VMEM((NG, H), F32),
            pltpu.VMEM((NG, 1), jnp.int32),
            pltpu.VMEM((NG, 1), jnp.int32),
        ],
    )(e, g1, g2, src3, batch2, w3, wpre, bpre.reshape(1, H), wout,
      bout.reshape(1, H))


def _sc_gather_body(src2_ref, dst2_ref, a_ref, b_ref, g1_ref, g2_ref,
                    idxs_ref, idxd_ref, ra0, ra1, rb0, rb1,
                    semg0, semg1, semw0, semw1):
    c = lax.axis_index("c")
    s = lax.axis_index("s")
    wid = s * NC + c
    base = wid * EPW
    pltpu.sync_copy(src2_ref.at[wid], idxs_ref)
    pltpu.sync_copy(dst2_ref.at[wid], idxd_ref)

    ra = (ra0, ra1)
    rb = (rb0, rb1)
    semg = (semg0, semg1)
    semw = (semw0, semw1)

    def fire_g(i, slot):
        pltpu.async_copy(a_ref.at[idxs_ref.at[i]], ra[slot], semg[slot])
        pltpu.async_copy(b_ref.at[idxd_ref.at[i]], rb[slot], semg[slot])

    def drain_g(slot):
        pltpu.make_async_copy(a_ref.at[idxs_ref.at[0]], ra[slot], semg[slot]).wait()
        pltpu.make_async_copy(b_ref.at[idxd_ref.at[0]], rb[slot], semg[slot]).wait()

    def fire_w(i, slot):
        off = base + i * CH
        pltpu.async_copy(ra[slot], g1_ref.at[pl.ds(off, CH)], semw[slot])
        pltpu.async_copy(rb[slot], g2_ref.at[pl.ds(off, CH)], semw[slot])

    def drain_w(slot):
        pltpu.make_async_copy(ra[slot], g1_ref.at[pl.ds(base, CH)], semw[slot]).wait()
        pltpu.make_async_copy(rb[slot], g2_ref.at[pl.ds(base, CH)], semw[slot]).wait()

    fire_g(0, 0)

    def body(g, carry):
        i0 = 2 * g
        i1 = i0 + 1

        @pl.when(g > 0)
        def _():
            drain_w(1)

        fire_g(i1, 1)
        drain_g(0)
        fire_w(i0, 0)

        @pl.when(g < NPAIR - 1)
        def _():
            drain_w(0)
            fire_g(i0 + 2, 0)

        drain_g(1)
        fire_w(i1, 1)
        return carry

    lax.fori_loop(0, NPAIR, body, 0)
    drain_w(0)
    fire_g(NCHUNK - 1, 0)
    drain_g(0)
    fire_w(NCHUNK - 1, 0)
    drain_w(1)
    drain_w(0)


@functools.lru_cache(maxsize=None)
def _sc_gather_kernel():
    return pl.kernel(
        _sc_gather_body,
        out_type=(jax.ShapeDtypeStruct((NE, H), F32),) * 2,
        mesh=plsc.VectorSubcoreMesh(**_MESH),
        scratch_types=[
            pltpu.VMEM((NCHUNK, CH), jnp.int32),
            pltpu.VMEM((NCHUNK, CH), jnp.int32),
            pltpu.VMEM((CH, H), F32),
            pltpu.VMEM((CH, H), F32),
            pltpu.VMEM((CH, H), F32),
            pltpu.VMEM((CH, H), F32),
            pltpu.SemaphoreType.DMA,
            pltpu.SemaphoreType.DMA,
            pltpu.SemaphoreType.DMA,
            pltpu.SemaphoreType.DMA,
        ],
    )


def _sc_gather(src2, dst2, a, b):
    return _sc_gather_kernel()(src2, dst2, a, b)


def _sc_scatter_body(m_ref, dst2_ref, zeros_ref, out_ref,
                     idxd_ref, mv0, mv1, agg_sh, seml0, seml1):
    c = lax.axis_index("c")
    s = lax.axis_index("s")
    wid = s * NC + c
    base = wid * EPW
    pltpu.sync_copy(dst2_ref.at[wid], idxd_ref)
    pltpu.sync_copy(zeros_ref, agg_sh.at[pl.ds(s * RPT, RPT)])
    plsc.subcore_barrier()

    mv = (mv0, mv1)
    seml = (seml0, seml1)

    def fire_l(i, slot):
        pltpu.async_copy(m_ref.at[pl.ds(base + i * CH, CH)], mv[slot], seml[slot])

    def drain_l(slot):
        pltpu.make_async_copy(m_ref.at[pl.ds(base, CH)], mv[slot], seml[slot]).wait()

    def scat(i, slot):
        pltpu.sync_copy(mv[slot], agg_sh.at[idxd_ref.at[i]], add=True)

    fire_l(0, 0)

    def body(g, carry):
        i0 = 2 * g
        i1 = i0 + 1
        fire_l(i1, 1)
        drain_l(0)
        scat(i0, 0)

        @pl.when(g < NPAIR - 1)
        def _():
            fire_l(i0 + 2, 0)

        drain_l(1)
        scat(i1, 1)
        return carry

    lax.fori_loop(0, NPAIR, body, 0)
    fire_l(NCHUNK - 1, 0)
    drain_l(0)
    scat(NCHUNK - 1, 0)
    plsc.subcore_barrier()
    pltpu.sync_copy(agg_sh.at[pl.ds(s * RPT, RPT)],
                    out_ref.at[c, pl.ds(s * RPT, RPT)])


@functools.lru_cache(maxsize=None)
def _sc_scatter_kernel():
    return pl.kernel(
        _sc_scatter_body,
        out_type=jax.ShapeDtypeStruct((NC, AGG_PAD, H), F32),
        mesh=plsc.VectorSubcoreMesh(**_MESH),
        scratch_types=[
            pltpu.VMEM((NCHUNK, CH), jnp.int32),
            pltpu.VMEM((CH, H), F32),
            pltpu.VMEM((CH, H), F32),
            pltpu.VMEM_SHARED((AGG_PAD, H), F32),
            pltpu.SemaphoreType.DMA,
            pltpu.SemaphoreType.DMA,
        ],
    )


def _sc_scatter(m, dst2, zeros_rt):
    return _sc_scatter_kernel()(m, dst2, zeros_rt)


def kernel(x, edge_index, edge_attr, batch, enc_Wn, enc_bn, enc_We, enc_be,
           W_msg, b_msg, W_node, b_node, W_pre, b_pre, W_out, b_out):
    src = edge_index[0]
    src2 = src.reshape(NW, NCHUNK, CH)
    dst2 = edge_index[1].reshape(NW, NCHUNK, CH)

    h = _linear(x, enc_Wn, enc_bn, relu=True, bm=2000)
    e = _linear(edge_attr, enc_We, enc_be, relu=True, bm=2000)
    zeros_rt = jnp.zeros((RPT, H), F32)

    out = None
    for l in range(NL):
        w1 = W_msg[l, :H]
        w2 = W_msg[l, H:2 * H]
        w3 = W_msg[l, 2 * H:]
        a, b = _ab(h, w1, w2, b_msg[l])
        g1, g2 = _sc_gather(src2, dst2, a, b)
        if l < NL - 1:
            e = _msg(e, g1, g2, w3)
            agg = _sc_scatter(e, dst2, zeros_rt)
            h = _node(h, agg, W_node[l, :H], W_node[l, H:], b_node[l])
        else:
            out = _readout(e, g1, g2, src.reshape(NE // _BE_RD, 1, _BE_RD),
                           batch.reshape(1, NN), w3, W_pre, b_pre, W_out, b_out)
    return out


# speedup vs baseline: 3.8502x; 1.0259x over previous
import functools

import jax
import jax.numpy as jnp
from jax import lax
from jax.experimental import pallas as pl
from jax.experimental.pallas import tpu as pltpu
from jax.experimental.pallas import tpu_sc as plsc

NN = 10000
NE = 320000
DE = 16
H = 128
NG = 64
NL = 4

NC, NS = 2, 16
NW = NC * NS
NEH = NE // 2
EPW2 = NEH // NW
CH2 = 40
NCHUNK2 = EPW2 // CH2
NPAIR2 = NCHUNK2 // 2
RPT = 640
AGG_PAD = NS * RPT

F32 = jnp.float32

_MESH = dict(core_axis_name="c", subcore_axis_name="s",
             num_cores=NC, num_subcores=NS)


def _linear_body(x_ref, w_ref, b_ref, o_ref, *, relu):
    y = jnp.dot(x_ref[...], w_ref[...], preferred_element_type=F32) + b_ref[...]
    if relu:
        y = jnp.maximum(y, 0.0)
    o_ref[...] = y


def _linear(x, w, b, *, relu, bm, nrows=None, row_off=0):
    m, k = x.shape
    n = w.shape[1]
    nrows = m if nrows is None else nrows
    ob = row_off // bm
    return pl.pallas_call(
        functools.partial(_linear_body, relu=relu),
        grid=(nrows // bm,),
        in_specs=[
            pl.BlockSpec((bm, k), lambda i: (ob + i, 0)),
            pl.BlockSpec((k, n), lambda i: (0, 0)),
            pl.BlockSpec((1, n), lambda i: (0, 0)),
        ],
        out_specs=pl.BlockSpec((bm, n), lambda i: (i, 0)),
        out_shape=jax.ShapeDtypeStruct((nrows, n), F32),
    )(x, w, b.reshape(1, n))


def _ab_body(h_ref, w1_ref, w2_ref, b1_ref, a_ref, b_ref):
    hb = h_ref[...]
    a_ref[...] = jnp.dot(hb, w1_ref[...], preferred_element_type=F32) + b1_ref[...]
    b_ref[...] = jnp.dot(hb, w2_ref[...], preferred_element_type=F32)


def _ab(h, w1, w2, bmsg, bm=2000):
    return pl.pallas_call(
        _ab_body,
        grid=(NN // bm,),
        in_specs=[
            pl.BlockSpec((bm, H), lambda i: (i, 0)),
            pl.BlockSpec((H, H), lambda i: (0, 0)),
            pl.BlockSpec((H, H), lambda i: (0, 0)),
            pl.BlockSpec((1, H), lambda i: (0, 0)),
        ],
        out_specs=[pl.BlockSpec((bm, H), lambda i: (i, 0))] * 2,
        out_shape=[jax.ShapeDtypeStruct((NN, H), F32)] * 2,
    )(h, w1, w2, bmsg.reshape(1, H))


def _msg_body(e_ref, g1_ref, g2_ref, w3_ref, o_ref):
    y = jnp.dot(e_ref[...], w3_ref[...], preferred_element_type=F32)
    o_ref[...] = jnp.maximum(y + g1_ref[...] + g2_ref[...], 0.0)


def _msg(e, g1, g2, w3, be=1280):
    ne = e.shape[0]
    return pl.pallas_call(
        _msg_body,
        grid=(ne // be,),
        in_specs=[
            pl.BlockSpec((be, H), lambda i: (i, 0)),
            pl.BlockSpec((be, H), lambda i: (i, 0)),
            pl.BlockSpec((be, H), lambda i: (i, 0)),
            pl.BlockSpec((H, H), lambda i: (0, 0)),
        ],
        out_specs=pl.BlockSpec((be, H), lambda i: (i, 0)),
        out_shape=jax.ShapeDtypeStruct((ne, H), F32),
    )(e, g1, g2, w3)


def _node_body(h_ref, a00_ref, a01_ref, a10_ref, a11_ref, wn1_ref, wn2_ref,
               b_ref, o_ref):
    agg = a00_ref[0] + a01_ref[0] + a10_ref[0] + a11_ref[0]
    y = jnp.dot(h_ref[...], wn1_ref[...], preferred_element_type=F32)
    y = y + jnp.dot(agg, wn2_ref[...], preferred_element_type=F32) + b_ref[...]
    o_ref[...] = jnp.maximum(y, 0.0)


def _node(h, p0, p1, wn1, wn2, bnode, bm=2000):
    wspec = pl.BlockSpec((H, H), lambda i: (0, 0))
    return pl.pallas_call(
        _node_body,
        grid=(NN // bm,),
        in_specs=[
            pl.BlockSpec((bm, H), lambda i: (i, 0)),
            pl.BlockSpec((1, bm, H), lambda i: (0, i, 0)),
            pl.BlockSpec((1, bm, H), lambda i: (1, i, 0)),
            pl.BlockSpec((1, bm, H), lambda i: (0, i, 0)),
            pl.BlockSpec((1, bm, H), lambda i: (1, i, 0)),
            wspec,
            wspec,
            pl.BlockSpec((1, H), lambda i: (0, 0)),
        ],
        out_specs=pl.BlockSpec((bm, H), lambda i: (i, 0)),
        out_shape=jax.ShapeDtypeStruct((NN, H), F32),
    )(h, p0, p0, p1, p1, wn1, wn2, bnode.reshape(1, H))


_BE_RD = 1280


def _rp_body(e_ref, g1_ref, g2_ref, src_ref, batch_ref, w3_ref, wpre_ref,
             bpre_ref, acc_ref, cnt_ref, lt_ref, le_ref):
    i = pl.program_id(0)

    @pl.when(i == 0)
    def _():
        acc_ref[...] = jnp.zeros_like(acc_ref)
        cnt_ref[...] = jnp.zeros_like(cnt_ref)
        brow = batch_ref[...]
        gcol = lax.broadcasted_iota(jnp.int32, (NG, 1), 0)
        lt_ref[...] = jnp.sum((brow < gcol).astype(jnp.int32), axis=1,
                              keepdims=True)
        le_ref[...] = jnp.sum((brow <= gcol).astype(jnp.int32), axis=1,
                              keepdims=True)

    m = jnp.dot(e_ref[...], w3_ref[...], preferred_element_type=F32)
    m = jnp.maximum(m + g1_ref[...] + g2_ref[...], 0.0)
    phi = jnp.maximum(
        jnp.dot(m, wpre_ref[...], preferred_element_type=F32) + bpre_ref[...], 0.0)
    srow = src_ref[0]
    onehot = ((srow >= lt_ref[...]) & (srow < le_ref[...])).astype(F32)
    acc_ref[...] += jnp.dot(onehot, phi, preferred_element_type=F32)
    cnt_ref[...] += jnp.dot(onehot, jnp.ones_like(phi), preferred_element_type=F32)


def _rp(e, g1, g2, src3, batch2, w3, wpre, bpre):
    nblk = e.shape[0] // _BE_RD
    return pl.pallas_call(
        _rp_body,
        grid=(nblk,),
        in_specs=[
            pl.BlockSpec((_BE_RD, H), lambda i: (i, 0)),
            pl.BlockSpec((_BE_RD, H), lambda i: (i, 0)),
            pl.BlockSpec((_BE_RD, H), lambda i: (i, 0)),
            pl.BlockSpec((1, 1, _BE_RD), lambda i: (i, 0, 0)),
            pl.BlockSpec((1, NN), lambda i: (0, 0)),
            pl.BlockSpec((H, H), lambda i: (0, 0)),
            pl.BlockSpec((H, H), lambda i: (0, 0)),
            pl.BlockSpec((1, H), lambda i: (0, 0)),
        ],
        out_specs=[pl.BlockSpec((NG, H), lambda i: (0, 0))] * 2,
        out_shape=[jax.ShapeDtypeStruct((NG, H), F32)] * 2,
        scratch_shapes=[
            pltpu.VMEM((NG, 1), jnp.int32),
            pltpu.VMEM((NG, 1), jnp.int32),
        ],
    )(e, g1, g2, src3, batch2, w3, wpre, bpre.reshape(1, H))


def _combine_body(a0_ref, a1_ref, c0_ref, c1_ref, wout_ref, bout_ref, o_ref):
    sums = a0_ref[...] + a1_ref[...]
    cnts = c0_ref[...] + c1_ref[...]
    pooled = sums / jnp.maximum(cnts, 1.0)
    o_ref[...] = (jnp.dot(pooled, wout_ref[...], preferred_element_type=F32)
                  + bout_ref[...])


def _combine(a0, a1, c0, c1, wout, bout):
    return pl.pallas_call(
        _combine_body,
        out_shape=jax.ShapeDtypeStruct((NG, H), F32),
    )(a0, a1, c0, c1, wout, bout.reshape(1, H))


def _gather_body(src2_ref, dst2_ref, a_ref, b_ref, g1_ref, g2_ref,
                 idxs_ref, idxd_ref, ra0, ra1, rb0, rb1,
                 semg0, semg1, semw0, semw1):
    c = lax.axis_index("c")
    s = lax.axis_index("s")
    wid = s * NC + c
    base = wid * EPW2
    pltpu.sync_copy(src2_ref.at[wid], idxs_ref)
    pltpu.sync_copy(dst2_ref.at[wid], idxd_ref)

    ra = (ra0, ra1)
    rb = (rb0, rb1)
    semg = (semg0, semg1)
    semw = (semw0, semw1)

    def fire_g(i, slot):
        pltpu.async_copy(a_ref.at[idxs_ref.at[i]], ra[slot], semg[slot])
        pltpu.async_copy(b_ref.at[idxd_ref.at[i]], rb[slot], semg[slot])

    def drain_g(slot):
        pltpu.make_async_copy(a_ref.at[idxs_ref.at[0]], ra[slot], semg[slot]).wait()
        pltpu.make_async_copy(b_ref.at[idxd_ref.at[0]], rb[slot], semg[slot]).wait()

    def fire_w(i, slot):
        off = base + i * CH2
        pltpu.async_copy(ra[slot], g1_ref.at[pl.ds(off, CH2)], semw[slot])
        pltpu.async_copy(rb[slot], g2_ref.at[pl.ds(off, CH2)], semw[slot])

    def drain_w(slot):
        pltpu.make_async_copy(ra[slot], g1_ref.at[pl.ds(base, CH2)], semw[slot]).wait()
        pltpu.make_async_copy(rb[slot], g2_ref.at[pl.ds(base, CH2)], semw[slot]).wait()

    fire_g(0, 0)

    def body(g, carry):
        i0 = 2 * g
        i1 = i0 + 1

        @pl.when(g > 0)
        def _():
            drain_w(1)

        fire_g(i1, 1)
        drain_g(0)
        fire_w(i0, 0)

        @pl.when(g < NPAIR2 - 1)
        def _():
            drain_w(0)
            fire_g(i0 + 2, 0)

        drain_g(1)
        fire_w(i1, 1)
        return carry

    lax.fori_loop(0, NPAIR2, body, 0)
    drain_w(0)
    fire_g(NCHUNK2 - 1, 0)
    drain_g(0)
    fire_w(NCHUNK2 - 1, 0)
    drain_w(1)
    drain_w(0)


@functools.lru_cache(maxsize=None)
def _sc_gather_kernel():
    return pl.kernel(
        _gather_body,
        out_type=(jax.ShapeDtypeStruct((NEH, H), F32),) * 2,
        mesh=plsc.VectorSubcoreMesh(**_MESH),
        scratch_types=[
            pltpu.VMEM((NCHUNK2, CH2), jnp.int32),
            pltpu.VMEM((NCHUNK2, CH2), jnp.int32),
            pltpu.VMEM((CH2, H), F32),
            pltpu.VMEM((CH2, H), F32),
            pltpu.VMEM((CH2, H), F32),
            pltpu.VMEM((CH2, H), F32),
            pltpu.SemaphoreType.DMA,
            pltpu.SemaphoreType.DMA,
            pltpu.SemaphoreType.DMA,
            pltpu.SemaphoreType.DMA,
        ],
    )


def _sc_gather(src2, dst2, a, b):
    return _sc_gather_kernel()(src2, dst2, a, b)


def _scatter_body(m_ref, dst2_ref, zeros_ref, out_ref,
                  idxd_ref, mv0, mv1, agg_sh, seml0, seml1):
    c = lax.axis_index("c")
    s = lax.axis_index("s")
    wid = s * NC + c
    base = wid * EPW2
    pltpu.sync_copy(dst2_ref.at[wid], idxd_ref)
    pltpu.sync_copy(zeros_ref, agg_sh.at[pl.ds(s * RPT, RPT)])
    plsc.subcore_barrier()

    mv = (mv0, mv1)
    seml = (seml0, seml1)

    def fire_l(i, slot):
        pltpu.async_copy(m_ref.at[pl.ds(base + i * CH2, CH2)], mv[slot], seml[slot])

    def drain_l(slot):
        pltpu.make_async_copy(m_ref.at[pl.ds(base, CH2)], mv[slot], seml[slot]).wait()

    def scat(i, slot):
        pltpu.sync_copy(mv[slot], agg_sh.at[idxd_ref.at[i]], add=True)

    fire_l(0, 0)

    def body(g, carry):
        i0 = 2 * g
        i1 = i0 + 1
        fire_l(i1, 1)
        drain_l(0)
        scat(i0, 0)

        @pl.when(g < NPAIR2 - 1)
        def _():
            fire_l(i0 + 2, 0)

        drain_l(1)
        scat(i1, 1)
        return carry

    lax.fori_loop(0, NPAIR2, body, 0)
    fire_l(NCHUNK2 - 1, 0)
    drain_l(0)
    scat(NCHUNK2 - 1, 0)
    plsc.subcore_barrier()
    pltpu.sync_copy(agg_sh.at[pl.ds(s * RPT, RPT)],
                    out_ref.at[c, pl.ds(s * RPT, RPT)])


@functools.lru_cache(maxsize=None)
def _sc_scatter_kernel():
    return pl.kernel(
        _scatter_body,
        out_type=jax.ShapeDtypeStruct((NC, AGG_PAD, H), F32),
        mesh=plsc.VectorSubcoreMesh(**_MESH),
        scratch_types=[
            pltpu.VMEM((NCHUNK2, CH2), jnp.int32),
            pltpu.VMEM((CH2, H), F32),
            pltpu.VMEM((CH2, H), F32),
            pltpu.VMEM_SHARED((AGG_PAD, H), F32),
            pltpu.SemaphoreType.DMA,
            pltpu.SemaphoreType.DMA,
        ],
    )


def _sc_scatter(m, dst2, zeros_rt):
    return _sc_scatter_kernel()(m, dst2, zeros_rt)


def kernel(x, edge_index, edge_attr, batch, enc_Wn, enc_bn, enc_We, enc_be,
           W_msg, b_msg, W_node, b_node, W_pre, b_pre, W_out, b_out):
    src = edge_index[0]
    dst = edge_index[1]
    srcH = src.reshape(2, NW, NCHUNK2, CH2)
    dstH = dst.reshape(2, NW, NCHUNK2, CH2)
    srcR = src.reshape(2, NEH // _BE_RD, 1, _BE_RD)
    batch2 = batch.reshape(1, NN)

    hn = _linear(x, enc_Wn, enc_bn, relu=True, bm=2000)
    e = [
        _linear(edge_attr, enc_We, enc_be, relu=True, bm=2000,
                nrows=NEH, row_off=0),
        _linear(edge_attr, enc_We, enc_be, relu=True, bm=2000,
                nrows=NEH, row_off=NEH),
    ]
    zeros_rt = jnp.zeros((RPT, H), F32)

    out = None
    for l in range(NL):
        w1 = W_msg[l, :H]
        w2 = W_msg[l, H:2 * H]
        w3 = W_msg[l, 2 * H:]
        a, b = _ab(hn, w1, w2, b_msg[l])
        g0 = _sc_gather(srcH[0], dstH[0], a, b)
        g1 = _sc_gather(srcH[1], dstH[1], a, b)
        if l < NL - 1:
            m0 = _msg(e[0], g0[0], g0[1], w3)
            p0 = _sc_scatter(m0, dstH[0], zeros_rt)
            m1 = _msg(e[1], g1[0], g1[1], w3)
            p1 = _sc_scatter(m1, dstH[1], zeros_rt)
            e = [m0, m1]
            hn = _node(hn, p0, p1, W_node[l, :H], W_node[l, H:], b_node[l])
        else:
            acc0, cnt0 = _rp(e[0], g0[0], g0[1], srcR[0], batch2, w3,
                             W_pre, b_pre)
            acc1, cnt1 = _rp(e[1], g1[0], g1[1], srcR[1], batch2, w3,
                             W_pre, b_pre)
            out = _combine(acc0, acc1, cnt0, cnt1, W_out, b_out)
    return out


# speedup vs baseline: 4.4182x; 1.1475x over previous
import functools

import jax
import jax.numpy as jnp
from jax import lax
from jax.experimental import pallas as pl
from jax.experimental.pallas import tpu as pltpu
from jax.experimental.pallas import tpu_sc as plsc

NN = 10000
NE = 320000
DE = 16
H = 128
NG = 64
NL = 4

NC, NS = 2, 16
NW = NC * NS
EPW = NE // NW
CH = 80
NCHUNK = EPW // CH
NPAIR = NCHUNK // 2
RPT = 640
AGG_PAD = NS * RPT

F32 = jnp.float32

_MESH = dict(core_axis_name="c", subcore_axis_name="s",
             num_cores=NC, num_subcores=NS)


def _linear_body(x_ref, w_ref, b_ref, o_ref, *, relu):
    y = jnp.dot(x_ref[...], w_ref[...], preferred_element_type=F32) + b_ref[...]
    if relu:
        y = jnp.maximum(y, 0.0)
    o_ref[...] = y


def _linear(x, w, b, *, relu, bm):
    m, k = x.shape
    n = w.shape[1]
    return pl.pallas_call(
        functools.partial(_linear_body, relu=relu),
        grid=(m // bm,),
        in_specs=[
            pl.BlockSpec((bm, k), lambda i: (i, 0)),
            pl.BlockSpec((k, n), lambda i: (0, 0)),
            pl.BlockSpec((1, n), lambda i: (0, 0)),
        ],
        out_specs=pl.BlockSpec((bm, n), lambda i: (i, 0)),
        out_shape=jax.ShapeDtypeStruct((m, n), F32),
    )(x, w, b.reshape(1, n))


def _ab_body(h_ref, w1_ref, w2_ref, a_ref, b_ref):
    hb = h_ref[...]
    a_ref[...] = jnp.dot(hb, w1_ref[...], preferred_element_type=F32)
    b_ref[...] = jnp.dot(hb, w2_ref[...], preferred_element_type=F32)


def _ab(h, w1, w2, bm=2000):
    return pl.pallas_call(
        _ab_body,
        grid=(NN // bm,),
        in_specs=[
            pl.BlockSpec((bm, H), lambda i: (i, 0)),
            pl.BlockSpec((H, H), lambda i: (0, 0)),
            pl.BlockSpec((H, H), lambda i: (0, 0)),
        ],
        out_specs=[pl.BlockSpec((bm, H), lambda i: (i, 0))] * 2,
        out_shape=[jax.ShapeDtypeStruct((NN, H), F32)] * 2,
    )(h, w1, w2)


def _node_body(h_ref, a0_ref, a1_ref, wn1_ref, wn2_ref, b_ref, o_ref):
    agg = a0_ref[0] + a1_ref[0]
    y = jnp.dot(h_ref[...], wn1_ref[...], preferred_element_type=F32)
    y = y + jnp.dot(agg, wn2_ref[...], preferred_element_type=F32) + b_ref[...]
    o_ref[...] = jnp.maximum(y, 0.0)


def _node(h, p, wn1, wn2, bnode, bm=2000):
    return pl.pallas_call(
        _node_body,
        grid=(NN // bm,),
        in_specs=[
            pl.BlockSpec((bm, H), lambda i: (i, 0)),
            pl.BlockSpec((1, bm, H), lambda i: (0, i, 0)),
            pl.BlockSpec((1, bm, H), lambda i: (1, i, 0)),
            pl.BlockSpec((H, H), lambda i: (0, 0)),
            pl.BlockSpec((H, H), lambda i: (0, 0)),
            pl.BlockSpec((1, H), lambda i: (0, 0)),
        ],
        out_specs=pl.BlockSpec((bm, H), lambda i: (i, 0)),
        out_shape=jax.ShapeDtypeStruct((NN, H), F32),
    )(h, p, p, wn1, wn2, bnode.reshape(1, H))


_BE_RD = 1280


def _readout_body(m_ref, src_ref, batch_ref, wpre_ref, bpre_ref,
                  wout_ref, bout_ref, o_ref, acc_ref, cnt_ref, lt_ref, le_ref):
    i = pl.program_id(0)

    @pl.when(i == 0)
    def _():
        acc_ref[...] = jnp.zeros_like(acc_ref)
        cnt_ref[...] = jnp.zeros_like(cnt_ref)
        brow = batch_ref[...]
        gcol = lax.broadcasted_iota(jnp.int32, (NG, 1), 0)
        lt_ref[...] = jnp.sum((brow < gcol).astype(jnp.int32), axis=1,
                              keepdims=True)
        le_ref[...] = jnp.sum((brow <= gcol).astype(jnp.int32), axis=1,
                              keepdims=True)

    phi = jnp.maximum(
        jnp.dot(m_ref[...], wpre_ref[...], preferred_element_type=F32)
        + bpre_ref[...], 0.0)
    srow = src_ref[0]
    onehot = ((srow >= lt_ref[...]) & (srow < le_ref[...])).astype(F32)
    acc_ref[...] += jnp.dot(onehot, phi, preferred_element_type=F32)
    cnt_ref[...] += jnp.dot(onehot, jnp.ones_like(phi), preferred_element_type=F32)

    @pl.when(i == pl.num_programs(0) - 1)
    def _():
        pooled = acc_ref[...] / jnp.maximum(cnt_ref[...], 1.0)
        o_ref[...] = (jnp.dot(pooled, wout_ref[...], preferred_element_type=F32)
                      + bout_ref[...])


def _readout(m, src3, batch2, wpre, bpre, wout, bout):
    nblk = NE // _BE_RD
    return pl.pallas_call(
        _readout_body,
        grid=(nblk,),
        in_specs=[
            pl.BlockSpec((_BE_RD, H), lambda i: (i, 0)),
            pl.BlockSpec((1, 1, _BE_RD), lambda i: (i, 0, 0)),
            pl.BlockSpec((1, NN), lambda i: (0, 0)),
            pl.BlockSpec((H, H), lambda i: (0, 0)),
            pl.BlockSpec((1, H), lambda i: (0, 0)),
            pl.BlockSpec((H, H), lambda i: (0, 0)),
            pl.BlockSpec((1, H), lambda i: (0, 0)),
        ],
        out_specs=pl.BlockSpec((NG, H), lambda i: (0, 0)),
        out_shape=jax.ShapeDtypeStruct((NG, H), F32),
        scratch_shapes=[
            pltpu.VMEM((NG, H), F32),
            pltpu.VMEM((NG, H), F32),
            pltpu.VMEM((NG, 1), jnp.int32),
            pltpu.VMEM((NG, 1), jnp.int32),
        ],
    )(m, src3, batch2, wpre, bpre.reshape(1, H), wout, bout.reshape(1, H))


def _fused_impl(src2_ref, dst2_ref, a_ref, b_ref, c_ref, m_ref,
                idxs_ref, idxd_ref, ra, rb, cv, semg, semw):
    c = lax.axis_index("c")
    s = lax.axis_index("s")
    wid = s * NC + c
    base = wid * EPW
    pltpu.sync_copy(src2_ref.at[wid], idxs_ref)
    pltpu.sync_copy(dst2_ref.at[wid], idxd_ref)

    def fire_g(i, slot):
        off = base + i * CH
        pltpu.async_copy(a_ref.at[idxs_ref.at[i]], ra[slot], semg[slot])
        pltpu.async_copy(b_ref.at[idxd_ref.at[i]], rb[slot], semg[slot])
        pltpu.async_copy(c_ref.at[pl.ds(off, CH)], cv[slot], semg[slot])

    def drain_g(slot):
        pltpu.make_async_copy(a_ref.at[idxs_ref.at[0]], ra[slot], semg[slot]).wait()
        pltpu.make_async_copy(b_ref.at[idxd_ref.at[0]], rb[slot], semg[slot]).wait()
        pltpu.make_async_copy(c_ref.at[pl.ds(base, CH)], cv[slot], semg[slot]).wait()

    def compute(slot):
        rab, rbb, cvb = ra[slot], rb[slot], cv[slot]

        def row(r, carry):
            for j in range(H // 16):
                sl = pl.ds(j * 16, 16)
                rab[r, sl] = jnp.maximum(rab[r, sl] + rbb[r, sl] + cvb[r, sl],
                                         0.0)
            return carry

        lax.fori_loop(0, CH, row, 0)

    def fire_w(i, slot):
        pltpu.async_copy(ra[slot], m_ref.at[pl.ds(base + i * CH, CH)], semw[slot])

    def drain_w(slot):
        pltpu.make_async_copy(ra[slot], m_ref.at[pl.ds(base, CH)], semw[slot]).wait()

    def step(i, slot):
        drain_g(slot)
        compute(slot)
        fire_w(i, slot)

    fire_g(0, 0)

    def body(g, carry):
        i0 = 2 * g
        i1 = i0 + 1

        @pl.when(g > 0)
        def _():
            drain_w(1)

        fire_g(i1, 1)
        step(i0, 0)

        @pl.when(g < NPAIR - 1)
        def _():
            drain_w(0)
            fire_g(i0 + 2, 0)

        step(i1, 1)
        return carry

    lax.fori_loop(0, NPAIR, body, 0)
    drain_w(0)
    fire_g(NCHUNK - 1, 0)
    step(NCHUNK - 1, 0)
    drain_w(1)
    drain_w(0)


def _fused_plain_body(src2_ref, dst2_ref, a_ref, b_ref, c_ref,
                      m_ref, idxs_ref, idxd_ref,
                      ra0, ra1, rb0, rb1, cv0, cv1,
                      semg0, semg1, semw0, semw1):
    _fused_impl(src2_ref, dst2_ref, a_ref, b_ref, c_ref, m_ref,
                idxs_ref, idxd_ref, (ra0, ra1), (rb0, rb1), (cv0, cv1),
                (semg0, semg1), (semw0, semw1))


@functools.lru_cache(maxsize=None)
def _sc_fused_plain_kernel():
    return pl.kernel(
        _fused_plain_body,
        out_type=jax.ShapeDtypeStruct((NE, H), F32),
        mesh=plsc.VectorSubcoreMesh(**_MESH),
        scratch_types=[
            pltpu.VMEM((NCHUNK, CH), jnp.int32),
            pltpu.VMEM((NCHUNK, CH), jnp.int32),
            pltpu.VMEM((CH, H), F32),
            pltpu.VMEM((CH, H), F32),
            pltpu.VMEM((CH, H), F32),
            pltpu.VMEM((CH, H), F32),
            pltpu.VMEM((CH, H), F32),
            pltpu.VMEM((CH, H), F32),
            pltpu.SemaphoreType.DMA,
            pltpu.SemaphoreType.DMA,
            pltpu.SemaphoreType.DMA,
            pltpu.SemaphoreType.DMA,
        ],
    )


def _sc_fused_plain(src2, dst2, a, b, cc):
    return _sc_fused_plain_kernel()(src2, dst2, a, b, cc)


def _scatter_body(m_ref, dst2_ref, zeros_ref, out_ref,
                  idxd_ref, mv0, mv1, agg_sh, seml0, seml1):
    c = lax.axis_index("c")
    s = lax.axis_index("s")
    wid = s * NC + c
    base = wid * EPW
    pltpu.sync_copy(dst2_ref.at[wid], idxd_ref)
    pltpu.sync_copy(zeros_ref, agg_sh.at[pl.ds(s * RPT, RPT)])
    plsc.subcore_barrier()

    mv = (mv0, mv1)
    seml = (seml0, seml1)

    def fire_l(i, slot):
        pltpu.async_copy(m_ref.at[pl.ds(base + i * CH, CH)], mv[slot], seml[slot])

    def drain_l(slot):
        pltpu.make_async_copy(m_ref.at[pl.ds(base, CH)], mv[slot], seml[slot]).wait()

    def scat(i, slot):
        pltpu.sync_copy(mv[slot], agg_sh.at[idxd_ref.at[i]], add=True)

    fire_l(0, 0)

    def body(g, carry):
        i0 = 2 * g
        i1 = i0 + 1
        fire_l(i1, 1)
        drain_l(0)
        scat(i0, 0)

        @pl.when(g < NPAIR - 1)
        def _():
            fire_l(i0 + 2, 0)

        drain_l(1)
        scat(i1, 1)
        return carry

    lax.fori_loop(0, NPAIR, body, 0)
    fire_l(NCHUNK - 1, 0)
    drain_l(0)
    scat(NCHUNK - 1, 0)
    plsc.subcore_barrier()
    pltpu.sync_copy(agg_sh.at[pl.ds(s * RPT, RPT)],
                    out_ref.at[c, pl.ds(s * RPT, RPT)])


@functools.lru_cache(maxsize=None)
def _sc_scatter_kernel():
    return pl.kernel(
        _scatter_body,
        out_type=jax.ShapeDtypeStruct((NC, AGG_PAD, H), F32),
        mesh=plsc.VectorSubcoreMesh(**_MESH),
        scratch_types=[
            pltpu.VMEM((NCHUNK, CH), jnp.int32),
            pltpu.VMEM((CH, H), F32),
            pltpu.VMEM((CH, H), F32),
            pltpu.VMEM_SHARED((AGG_PAD, H), F32),
            pltpu.SemaphoreType.DMA,
            pltpu.SemaphoreType.DMA,
        ],
    )


def _sc_scatter(m, dst2, zeros_rt):
    return _sc_scatter_kernel()(m, dst2, zeros_rt)


def kernel(x, edge_index, edge_attr, batch, enc_Wn, enc_bn, enc_We, enc_be,
           W_msg, b_msg, W_node, b_node, W_pre, b_pre, W_out, b_out):
    src = edge_index[0]
    src2 = src.reshape(NW, NCHUNK, CH)
    dst2 = edge_index[1].reshape(NW, NCHUNK, CH)
    src3 = src.reshape(NE // _BE_RD, 1, _BE_RD)
    batch2 = batch.reshape(1, NN)

    hn = _linear(x, enc_Wn, enc_bn, relu=True, bm=2000)
    e = _linear(edge_attr, enc_We, enc_be, relu=True, bm=2000)
    zeros_rt = jnp.zeros((RPT, H), F32)

    out = None
    cc = _linear(e, W_msg[0, 2 * H:], b_msg[0], relu=False, bm=2000)
    for l in range(NL):
        a, b = _ab(hn, W_msg[l, :H], W_msg[l, H:2 * H])
        m = _sc_fused_plain(src2, dst2, a, b, cc)
        if l < NL - 1:
            cc = _linear(m, W_msg[l + 1, 2 * H:], b_msg[l + 1], relu=False,
                         bm=2000)
            p = _sc_scatter(m, dst2, zeros_rt)
            hn = _node(hn, p, W_node[l, :H], W_node[l, H:], b_node[l])
        else:
            out = _readout(m, src3, batch2, W_pre, b_pre, W_out, b_out)
    return out
